# Initial kernel scaffold; baseline (speedup 1.0000x reference)
import jax
import jax.numpy as jnp
from jax.experimental import pallas as pl


def kernel(height, event_info):
    raise NotImplementedError("write your pallas kernel here")


# speedup vs baseline: 2.3783x; 2.3783x over previous
import functools

import jax
import jax.numpy as jnp
from jax import lax
from jax.experimental import pallas as pl
from jax.experimental.pallas import tpu as pltpu
from jax.experimental.pallas import tpu_sc as plsc

ROWS = 128
N = 32768
NVREG = N // 16
BINS = 4096
NW = 32
ROWS_PER_W = ROWS // NW


def _sc_sort_rows(hbits_flat, ei_flat):
  mesh = plsc.VectorSubcoreMesh(core_axis_name="c", subcore_axis_name="s")

  @functools.partial(
      pl.kernel,
      out_type=jax.ShapeDtypeStruct((ROWS * N,), jnp.int32),
      mesh=mesh,
      scratch_types=[
          pltpu.VMEM((N,), jnp.int32),
          pltpu.VMEM((N,), jnp.int32),
          pltpu.VMEM((BINS,), jnp.int32),
      ],
      compiler_params=pltpu.CompilerParams(needs_layout_passes=False),
  )
  def sort_kernel(h_hbm, e_hbm, out_hbm, buf_a, buf_b, cnt):
    cid = lax.axis_index("c")
    sid = lax.axis_index("s")
    wid = sid * 2 + cid

    def do_row(r, _):
      base = (wid * ROWS_PER_W + r) * N
      pltpu.sync_copy(h_hbm.at[pl.ds(base, N)], buf_a)
      pltpu.sync_copy(e_hbm.at[pl.ds(base, N)], buf_b)

      def pack_body(i, _):
        hv = buf_a[pl.ds(i * 16, 16)]
        ev = buf_b[pl.ds(i * 16, 16)]
        buf_a[pl.ds(i * 16, 16)] = lax.shift_left(hv, 1) | ev
        return 0

      lax.fori_loop(0, NVREG, pack_body, 0)

      for p, shift in enumerate((1, 13, 25)):
        src = buf_a if p % 2 == 0 else buf_b
        dst = buf_b if p % 2 == 0 else buf_a

        def zero_body(i, _):
          cnt[pl.ds(i * 16, 16)] = jnp.zeros((16,), jnp.int32)
          return 0

        lax.fori_loop(0, BINS // 16, zero_body, 0)

        def hist_body(i, _, src=src, shift=shift):
          x = src[pl.ds(i * 16, 16)]
          d = (BINS - 1) - (lax.shift_right_logical(x, shift) & (BINS - 1))
          rc, last = plsc.scan_count(d)
          plsc.addupdate_scatter(cnt, [d], rc, mask=last)
          return 0

        lax.fori_loop(0, NVREG, hist_body, 0)

        def scan_body(i, carry):
          v = cnt[pl.ds(i * 16, 16)]
          s = plsc.cumsum(v)
          cnt[pl.ds(i * 16, 16)] = s - v + carry
          return carry + jnp.sum(v)

        lax.fori_loop(0, BINS // 16, scan_body, jnp.int32(0))

        def perm_body(i, _, src=src, dst=dst, shift=shift):
          x = src[pl.ds(i * 16, 16)]
          d = (BINS - 1) - (lax.shift_right_logical(x, shift) & (BINS - 1))
          rc, last = plsc.scan_count(d)
          dest = plsc.load_gather(cnt, [d]) + rc - 1
          plsc.store_scatter(dst, [dest], x)
          plsc.addupdate_scatter(cnt, [d], rc, mask=last)
          return 0

        lax.fori_loop(0, NVREG, perm_body, 0)

      pltpu.sync_copy(buf_b, out_hbm.at[pl.ds(base, N)])
      return 0

    lax.fori_loop(0, ROWS_PER_W, do_row, 0)

  return sort_kernel(hbits_flat, ei_flat)


def _tc_post(pk_sorted):
  blk = 8
  grid = ROWS // blk

  def post_kernel(pk_ref, et_ref, ei_ref, iv_ref, mk_ref, cf_ref):
    x = pk_ref[...]
    h = lax.bitcast_convert_type(lax.shift_right_logical(x, 1), jnp.float32)
    e = (x & 1).astype(jnp.float32)
    col = lax.broadcasted_iota(jnp.int32, (blk, N), 1)

    c = e
    for t in range(15):
      s = 1 << t
      c = c + jnp.where(col >= s, jnp.roll(c, s, axis=1), 0.0)

    lin = c + 1.0
    et_ref[...] = h
    ei_ref[...] = e
    iv_ref[...] = h - jnp.roll(h, -1, axis=1)
    mk_ref[...] = jnp.where(col < N - 1, e, 0.0)
    cf_ref[...] = lin * (lin - 1.0) * 0.5

  out_shape = jax.ShapeDtypeStruct((ROWS, N), jnp.float32)
  spec = pl.BlockSpec((blk, N), lambda i: (i, 0))
  return pl.pallas_call(
      post_kernel,
      grid=(grid,),
      in_specs=[pl.BlockSpec((blk, N), lambda i: (i, 0))],
      out_specs=[spec] * 5,
      out_shape=[out_shape] * 5,
  )(pk_sorted)


def kernel(height, event_info):
  hbits = lax.bitcast_convert_type(height, jnp.int32).reshape(-1)
  ei_i = event_info.astype(jnp.int32).reshape(-1)
  pk_sorted = _sc_sort_rows(hbits, ei_i).reshape(ROWS, N)
  et, ei, iv, mk, cf = _tc_post(pk_sorted)
  return (
      et,
      ei[:, :-1],
      iv[:, :-1],
      mk.astype(jnp.uint8),
      cf[:, :-1],
  )


# speedup vs baseline: 2.5097x; 1.0553x over previous
import functools

import jax
import jax.numpy as jnp
from jax import lax
from jax.experimental import pallas as pl
from jax.experimental.pallas import tpu as pltpu
from jax.experimental.pallas import tpu_sc as plsc

ROWS = 128
N = 32768
NW = 32
ROWS_PER_W = ROWS // NW
NCHUNK = 4
CHUNK = N // NCHUNK
CVREG = CHUNK // 16
CNT_WORDS = 2048

PASSES = ((1, 2048), (12, 2048), (23, 512))


def _sc_sort_rows(hbits_flat, ei_flat):
  mesh = plsc.VectorSubcoreMesh(core_axis_name="c", subcore_axis_name="s")

  @functools.partial(
      pl.kernel,
      out_type=jax.ShapeDtypeStruct((ROWS * N,), jnp.int32),
      mesh=mesh,
      scratch_types=[
          pltpu.VMEM((N,), jnp.int32),
          pltpu.VMEM((N,), jnp.int32),
      ] + [pltpu.VMEM((CNT_WORDS,), jnp.int32) for _ in range(NCHUNK)],
      compiler_params=pltpu.CompilerParams(needs_layout_passes=False),
  )
  def sort_kernel(h_hbm, e_hbm, out_hbm, buf_a, buf_b, c0, c1, c2, c3):
    cnts = (c0, c1, c2, c3)
    cid = lax.axis_index("c")
    sid = lax.axis_index("s")
    wid = sid * 2 + cid

    def do_row(r, _):
      base = (wid * ROWS_PER_W + r) * N
      pltpu.sync_copy(h_hbm.at[pl.ds(base, N)], buf_a)
      pltpu.sync_copy(e_hbm.at[pl.ds(base, N)], buf_b)

      for p, (shift, bins) in enumerate(PASSES):
        src = buf_a if p % 2 == 0 else buf_b
        dst = buf_b if p % 2 == 0 else buf_a
        mask = bins - 1
        nvec = bins // 16

        def zero_body(i, _):
          z = jnp.zeros((16,), jnp.int32)
          for c in range(NCHUNK):
            cnts[c][pl.ds(i * 16, 16)] = z
          return 0

        lax.fori_loop(0, nvec, zero_body, 0)

        if p == 0:
          def hist_body(i, _, shift=shift, mask=mask):
            for c in range(NCHUNK):
              off = c * CHUNK + i * 16
              hv = buf_a[pl.ds(off, 16)]
              ev = buf_b[pl.ds(off, 16)]
              x = lax.shift_left(hv, 1) | ev
              buf_a[pl.ds(off, 16)] = x
              d = mask - (lax.shift_right_logical(x, shift) & mask)
              rc, last = plsc.scan_count(d)
              plsc.addupdate_scatter(cnts[c], [d], rc, mask=last)
            return 0
        else:
          def hist_body(i, _, src=src, shift=shift, mask=mask):
            for c in range(NCHUNK):
              off = c * CHUNK + i * 16
              x = src[pl.ds(off, 16)]
              d = mask - (lax.shift_right_logical(x, shift) & mask)
              rc, last = plsc.scan_count(d)
              plsc.addupdate_scatter(cnts[c], [d], rc, mask=last)
            return 0

        lax.fori_loop(0, CVREG, hist_body, 0)

        def scan_body(i, carry):
          ds_ = pl.ds(i * 16, 16)
          v = [cnts[c][ds_] for c in range(NCHUNK)]
          t = v[0] + v[1] + v[2] + v[3]
          s = plsc.cumsum(t)
          b = s - t + carry
          cnts[0][ds_] = b
          b = b + v[0]
          cnts[1][ds_] = b
          b = b + v[1]
          cnts[2][ds_] = b
          b = b + v[2]
          cnts[3][ds_] = b
          return carry + jnp.sum(t)

        lax.fori_loop(0, nvec, scan_body, jnp.int32(0))

        def perm_body(i, _, src=src, dst=dst, shift=shift, mask=mask):
          for c in range(NCHUNK):
            off = c * CHUNK + i * 16
            x = src[pl.ds(off, 16)]
            d = mask - (lax.shift_right_logical(x, shift) & mask)
            rc, last = plsc.scan_count(d)
            dest = plsc.load_gather(cnts[c], [d]) + rc - 1
            plsc.store_scatter(dst, [dest], x)
            plsc.addupdate_scatter(cnts[c], [d], rc, mask=last)
          return 0

        lax.fori_loop(0, CVREG, perm_body, 0)

      pltpu.sync_copy(buf_b, out_hbm.at[pl.ds(base, N)])
      return 0

    lax.fori_loop(0, ROWS_PER_W, do_row, 0)

  return sort_kernel(hbits_flat, ei_flat)


def _tc_post(pk_sorted):
  blk = 8
  grid = ROWS // blk

  def post_kernel(pk_ref, et_ref, ei_ref, iv_ref, mk_ref, cf_ref):
    x = pk_ref[...]
    h = lax.bitcast_convert_type(lax.shift_right_logical(x, 1), jnp.float32)
    e = (x & 1).astype(jnp.float32)
    col = lax.broadcasted_iota(jnp.int32, (blk, N), 1)

    c = e
    for t in range(15):
      s = 1 << t
      c = c + jnp.where(col >= s, jnp.roll(c, s, axis=1), 0.0)

    lin = c + 1.0
    et_ref[...] = h
    ei_ref[...] = e
    iv_ref[...] = h - jnp.roll(h, -1, axis=1)
    mk_ref[...] = jnp.where(col < N - 1, e, 0.0)
    cf_ref[...] = lin * (lin - 1.0) * 0.5

  out_shape = jax.ShapeDtypeStruct((ROWS, N), jnp.float32)
  spec = pl.BlockSpec((blk, N), lambda i: (i, 0))
  return pl.pallas_call(
      post_kernel,
      grid=(grid,),
      in_specs=[pl.BlockSpec((blk, N), lambda i: (i, 0))],
      out_specs=[spec] * 5,
      out_shape=[out_shape] * 5,
  )(pk_sorted)


def kernel(height, event_info):
  hbits = lax.bitcast_convert_type(height, jnp.int32).reshape(-1)
  ei_i = event_info.astype(jnp.int32).reshape(-1)
  pk_sorted = _sc_sort_rows(hbits, ei_i).reshape(ROWS, N)
  et, ei, iv, mk, cf = _tc_post(pk_sorted)
  return (
      et,
      ei[:, :-1],
      iv[:, :-1],
      mk.astype(jnp.uint8),
      cf[:, :-1],
  )
